# Initial kernel scaffold; baseline (speedup 1.0000x reference)
import jax
import jax.numpy as jnp
from jax.experimental import pallas as pl


def kernel(batch, emb, Wih_f, Whh_f, bih_f, bhh_f, Wih_b, Whh_b, bih_b, bhh_b, W_out, b_out):
    raise NotImplementedError("write your pallas kernel here")


# speedup vs baseline: 1.3167x; 1.3167x over previous
import functools

import jax
import jax.numpy as jnp
from jax import lax
from jax.experimental import pallas as pl
from jax.experimental.pallas import tpu as pltpu
from jax.experimental.pallas import tpu_sc as plsc


def _make_sc_gather(V, D, N):
    info = plsc.get_sparse_core_info()
    NC, NS = info.num_cores, info.num_subcores
    NW = NC * NS
    assert N % NW == 0
    n_per_w = N // NW
    CHUNK = 800
    assert n_per_w % CHUNK == 0
    n_chunks = n_per_w // CHUNK

    mesh = plsc.VectorSubcoreMesh(core_axis_name="c", subcore_axis_name="s")

    @functools.partial(
        pl.kernel,
        out_type=jax.ShapeDtypeStruct((N, D), jnp.float32),
        mesh=mesh,
        scratch_types=[
            pltpu.VMEM((n_per_w,), jnp.int32),
            pltpu.VMEM((CHUNK, D), jnp.float32),
            pltpu.SemaphoreType.DMA,
        ],
        compiler_params=pltpu.CompilerParams(use_tc_tiling_on_sc=False),
    )
    def gather(table_hbm, idx_hbm, out_hbm, idx_v, rows_v, sem):
        wid = lax.axis_index("s") * NC + lax.axis_index("c")
        base = wid * n_per_w
        pltpu.sync_copy(idx_hbm.at[pl.ds(base, n_per_w)], idx_v)
        for c in range(n_chunks):
            off = c * CHUNK
            pltpu.async_copy(
                table_hbm.at[idx_v.at[pl.ds(off, CHUNK)]], rows_v, sem
            ).wait()
            pltpu.sync_copy(rows_v, out_hbm.at[pl.ds(base + off, CHUNK)])

    return gather


def _lstm_body(H, L, xf_ref, xb_ref, wih_f, whh_f, b_f, wih_b, whh_b, b_b,
               hsf_ref, hsb_ref, hf, cf, hb, cb):
    t = pl.program_id(0)

    @pl.when(t == 0)
    def _():
        hf[...] = jnp.zeros_like(hf)
        cf[...] = jnp.zeros_like(cf)
        hb[...] = jnp.zeros_like(hb)
        cb[...] = jnp.zeros_like(cb)

    def step(x, w_ih, w_hh, b, h_s, c_s, out_ref):
        gates = (
            jnp.dot(x, w_ih[...], preferred_element_type=jnp.float32)
            + jnp.dot(h_s[...], w_hh[...], preferred_element_type=jnp.float32)
            + b[...]
        )
        i = jax.nn.sigmoid(gates[:, 0 * H:1 * H])
        f = jax.nn.sigmoid(gates[:, 1 * H:2 * H])
        g = jnp.tanh(gates[:, 2 * H:3 * H])
        o = jax.nn.sigmoid(gates[:, 3 * H:4 * H])
        c = f * c_s[...] + i * g
        h = o * jnp.tanh(c)
        c_s[...] = c
        h_s[...] = h
        out_ref[0] = h

    step(xf_ref[0], wih_f, whh_f, b_f, hf, cf, hsf_ref)
    step(xb_ref[0], wih_b, whh_b, b_b, hb, cb, hsb_ref)


def _run_lstm(x, wih_f_T, whh_f_T, b_f, wih_b_T, whh_b_T, b_b, interpret=False):
    L, B, E = x.shape
    H = whh_f_T.shape[0]
    const = lambda shape: pl.BlockSpec(shape, lambda t: (0,) * len(shape))
    return pl.pallas_call(
        functools.partial(_lstm_body, H, L),
        grid=(L,),
        in_specs=[
            pl.BlockSpec((1, B, E), lambda t: (t, 0, 0)),
            pl.BlockSpec((1, B, E), lambda t: (L - 1 - t, 0, 0)),
            const((E, 4 * H)), const((H, 4 * H)), const((1, 4 * H)),
            const((E, 4 * H)), const((H, 4 * H)), const((1, 4 * H)),
        ],
        out_specs=[
            pl.BlockSpec((1, B, H), lambda t: (t, 0, 0)),
            pl.BlockSpec((1, B, H), lambda t: (L - 1 - t, 0, 0)),
        ],
        out_shape=[
            jax.ShapeDtypeStruct((L, B, H), jnp.float32),
            jax.ShapeDtypeStruct((L, B, H), jnp.float32),
        ],
        scratch_shapes=[pltpu.VMEM((B, H), jnp.float32)] * 4,
        compiler_params=pltpu.CompilerParams(
            dimension_semantics=("arbitrary",),
        ),
        interpret=interpret,
    )(x, x, wih_f_T, whh_f_T, b_f, wih_b_T, whh_b_T, b_b)


def _attn_body(L, hsf_ref, hsb_ref, hnf_ref, hnb_ref, wof, wob, bo,
               out_ref, m, d, accf, accb):
    t = pl.program_id(0)

    @pl.when(t == 0)
    def _():
        m[...] = jnp.full_like(m, -jnp.inf)
        d[...] = jnp.zeros_like(d)
        accf[...] = jnp.zeros_like(accf)
        accb[...] = jnp.zeros_like(accb)

    hf = hsf_ref[0]
    hb = hsb_ref[0]
    s = (jnp.sum(hf * hnf_ref[0], axis=-1, keepdims=True)
         + jnp.sum(hb * hnb_ref[0], axis=-1, keepdims=True))
    m_new = jnp.maximum(m[...], s)
    alpha = jnp.exp(m[...] - m_new)
    p = jnp.exp(s - m_new)
    d[...] = d[...] * alpha + p
    accf[...] = accf[...] * alpha + p * hf
    accb[...] = accb[...] * alpha + p * hb
    m[...] = m_new

    @pl.when(t == L - 1)
    def _():
        inv = 1.0 / d[...]
        ctxf = accf[...] * inv
        ctxb = accb[...] * inv
        out_ref[...] = (
            jnp.dot(ctxf, wof[...], preferred_element_type=jnp.float32)
            + jnp.dot(ctxb, wob[...], preferred_element_type=jnp.float32)
            + bo[...]
        )


def _run_attn(hs_f, hs_b, wof, wob, bo, interpret=False):
    L, B, H = hs_f.shape
    C = wof.shape[1]
    const = lambda shape: pl.BlockSpec(shape, lambda t: (0,) * len(shape))
    return pl.pallas_call(
        functools.partial(_attn_body, L),
        grid=(L,),
        in_specs=[
            pl.BlockSpec((1, B, H), lambda t: (t, 0, 0)),
            pl.BlockSpec((1, B, H), lambda t: (t, 0, 0)),
            pl.BlockSpec((1, B, H), lambda t: (0, 0, 0)),
            pl.BlockSpec((1, B, H), lambda t: (L - 1, 0, 0)),
            const((H, C)), const((H, C)), const((1, C)),
        ],
        out_specs=pl.BlockSpec((B, C), lambda t: (0, 0)),
        out_shape=jax.ShapeDtypeStruct((B, C), jnp.float32),
        scratch_shapes=[
            pltpu.VMEM((B, 1), jnp.float32),
            pltpu.VMEM((B, 1), jnp.float32),
            pltpu.VMEM((B, H), jnp.float32),
            pltpu.VMEM((B, H), jnp.float32),
        ],
        compiler_params=pltpu.CompilerParams(
            dimension_semantics=("arbitrary",),
        ),
        interpret=interpret,
    )(hs_f, hs_b, hs_b, hs_f, wof, wob, bo)


def kernel(batch, emb, Wih_f, Whh_f, bih_f, bhh_f, Wih_b, Whh_b, bih_b, bhh_b,
           W_out, b_out):
    B, L = batch.shape
    V, E = emb.shape
    H = Whh_f.shape[1]
    C = W_out.shape[0]

    idx = batch.astype(jnp.int32).T.reshape(-1)
    x_flat = _make_sc_gather(V, E, L * B)(emb, idx)
    x = x_flat.reshape(L, B, E)

    b_f = (bih_f + bhh_f).reshape(1, 4 * H)
    b_b = (bih_b + bhh_b).reshape(1, 4 * H)
    hs_f, hs_b = _run_lstm(x, Wih_f.T, Whh_f.T, b_f, Wih_b.T, Whh_b.T, b_b)

    woutT = W_out.T
    out = _run_attn(hs_f, hs_b, woutT[:H], woutT[H:], b_out.reshape(1, C))
    return out
